# baseline (speedup 1.0000x reference)
import functools

import jax
import jax.numpy as jnp
from jax import lax
from jax.experimental import pallas as pl
from jax.experimental.pallas import tpu as pltpu
from jax.experimental.pallas import tpu_sc as plsc

NC = 2
NS = 16
NW = NC * NS
L = 16
H = 768
HC = H // L
HW = H // 2
HC2 = H // (2 * L)
R = 32
NB = 2
K = 4
EPS = 1e-12
MASK_HI = jnp.int32(-65536)


def _rsqrt(v):
    i = plsc.bitcast(v, jnp.int32)
    y = plsc.bitcast(jnp.int32(0x5F3759DF) - (i >> 1), jnp.float32)
    for _ in range(3):
        y = y * (1.5 - 0.5 * v * y * y)
    return y


def _make_kernel(n_rows):
    rows_per_w = n_rows // NW
    chunks = rows_per_w // R
    mesh = plsc.VectorSubcoreMesh(
        core_axis_name="c", subcore_axis_name="s",
        num_cores=NC, num_subcores=NS)

    @functools.partial(
        pl.kernel,
        out_type=jax.ShapeDtypeStruct((n_rows * H,), jnp.float32),
        mesh=mesh,
        compiler_params=pltpu.CompilerParams(needs_layout_passes=False),
        scratch_types=[
            pltpu.VMEM((rows_per_w,), jnp.int32),
            pltpu.VMEM((NB, R, HW), jnp.int32),
            pltpu.VMEM((NB, R, HW), jnp.int32),
            pltpu.VMEM((NB * R * H,), jnp.float32),
            pltpu.VMEM((R * L,), jnp.float32),
            pltpu.VMEM((R * L,), jnp.float32),
            pltpu.SMEM((R,), jnp.float32),
            pltpu.SMEM((R,), jnp.float32),
            pltpu.VMEM((H,), jnp.float32),
            pltpu.VMEM((H,), jnp.float32),
            pltpu.SemaphoreType.DMA((NB,)),
            pltpu.SemaphoreType.DMA((NB,)),
            pltpu.SemaphoreType.DMA((NB,)),
            pltpu.SemaphoreType.DMA,
        ],
    )
    def kern(emb_hbm, ids_hbm, tab_hbm, gam_hbm, bet_hbm, out_hbm,
             ids_v, p_v, y_v, o_v, sp_v, sq_v, a_sm, d_sm, g_v, b_v,
             sem_g, sem_e, sem_o, sem_i):
        wid = lax.axis_index("s") * NC + lax.axis_index("c")
        wbase = wid * rows_per_w
        pltpu.sync_copy(gam_hbm, g_v)
        pltpu.sync_copy(bet_hbm, b_v)
        pltpu.async_copy(ids_hbm.at[pl.ds(wbase, rows_per_w)], ids_v,
                         sem_i).wait()

        def start_loads(c, nb):
            idx = ids_v.at[pl.ds(c * R, R)]
            pltpu.async_copy(tab_hbm.at[idx], p_v.at[nb], sem_g.at[nb])
            pltpu.async_copy(emb_hbm.at[pl.ds(wbase + c * R, R)],
                             y_v.at[nb], sem_e.at[nb])

        start_loads(0, 0)

        iota2 = lax.iota(jnp.int32, L) * 2

        def chunk_body(c, _):
            nb = lax.rem(c, NB)

            idx = ids_v.at[pl.ds(c * R, R)]
            pltpu.make_async_copy(tab_hbm.at[idx], p_v.at[nb],
                                  sem_g.at[nb]).wait()
            pltpu.make_async_copy(emb_hbm.at[pl.ds(wbase + c * R, R)],
                                  y_v.at[nb], sem_e.at[nb]).wait()

            @pl.when(c + 1 < chunks)
            def _():
                start_loads(c + 1, lax.rem(c + 1, NB))

            @pl.when(c >= NB)
            def _():
                pltpu.make_async_copy(
                    o_v.at[pl.ds(nb * (R * H), R * H)],
                    out_hbm.at[pl.ds((wbase + (c - NB) * R) * H, R * H)],
                    sem_o.at[nb]).wait()

            RI = 2
            def row_body(q, _):
                r0 = q * RI
                def h_body(m, carry):
                    out = []
                    for i in range(RI):
                        s, ss, ix = carry[3 * i:3 * i + 3]
                        sl = pl.ds(m * L, L)
                        ew = y_v[nb, r0 + i, sl]
                        pw = p_v[nb, r0 + i, sl]
                        e0 = plsc.bitcast(ew << 16, jnp.float32)
                        e1 = plsc.bitcast(ew & MASK_HI, jnp.float32)
                        p0 = plsc.bitcast(pw << 16, jnp.float32)
                        p1 = plsc.bitcast(pw & MASK_HI, jnp.float32)
                        x0 = e0 + p0
                        x1 = e1 + p1
                        plsc.store_scatter(o_v, [ix], x0)
                        plsc.store_scatter(o_v, [ix + 1], x1)
                        out += [s + x0 + x1, ss + x0 * x0 + x1 * x1,
                                ix + 2 * L]
                    return tuple(out)
                z = jnp.zeros((L,), jnp.float32)
                carry0 = []
                for i in range(RI):
                    carry0 += [z, z, nb * (R * H) + (r0 + i) * H + iota2]
                acc = plsc.parallel_loop(
                    0, HC2, 1, unroll=4, carry=tuple(carry0))(h_body)
                for i in range(RI):
                    sp_v[pl.ds((r0 + i) * L, L)] = acc[3 * i]
                    sq_v[pl.ds((r0 + i) * L, L)] = acc[3 * i + 1]
                return 0
            lax.fori_loop(0, R // RI, row_body, 0)

            for k in range(R // L):
                rows16 = (lax.iota(jnp.int32, L) + k * L) * L
                s = jnp.zeros((L,), jnp.float32)
                ss = jnp.zeros((L,), jnp.float32)
                for j in range(L):
                    fidx = rows16 + j
                    s = s + plsc.load_gather(sp_v, [fidx])
                    ss = ss + plsc.load_gather(sq_v, [fidx])
                mean = s * (1.0 / H)
                var = ss * (1.0 / H) - mean * mean
                rstd = _rsqrt(var + EPS)
                nmr = -mean * rstd
                for j in range(L):
                    a_sm[k * L + j] = rstd[j]
                    d_sm[k * L + j] = nmr[j]

            def hb(h, _):
                sl = pl.ds(h * L, L)
                g = g_v[sl]
                b = b_v[sl]
                def rb(r):
                    sl2 = pl.ds(nb * (R * H) + r * H + h * L, L)
                    x = o_v[sl2]
                    o_v[sl2] = (x * a_sm[r] + d_sm[r]) * g + b
                plsc.parallel_loop(0, R, 1, unroll=8)(rb)
                return 0
            lax.fori_loop(0, HC, hb, 0)

            pltpu.async_copy(o_v.at[pl.ds(nb * (R * H), R * H)],
                             out_hbm.at[pl.ds((wbase + c * R) * H, R * H)],
                             sem_o.at[nb])
            return 0

        lax.fori_loop(0, chunks, chunk_body, 0)

        for j in range(NB):
            pltpu.make_async_copy(o_v.at[pl.ds(j * (R * H), R * H)],
                                  out_hbm.at[pl.ds(wbase * H, R * H)],
                                  sem_o.at[j]).wait()

    return kern


def _pack_bf16_words(x):
    bf = x.astype(jnp.bfloat16)
    return lax.bitcast_convert_type(
        bf.reshape(*bf.shape[:-1], bf.shape[-1] // 2, 2), jnp.int32)


def kernel(inputs_embeds, position_ids, pos_table, ln_gamma, ln_beta):
    b, s, h = inputs_embeds.shape
    n = b * s
    ids = position_ids.reshape(n).astype(jnp.int32)
    tab_i32 = _pack_bf16_words(pos_table)
    gam = ln_gamma.astype(jnp.float32)
    bet = ln_beta.astype(jnp.float32)
    emb = inputs_embeds.reshape(n, h)
    nk = n // K
    kern = _make_kernel(nk)
    outs = []
    for k in range(K):
        emb_k = _pack_bf16_words(emb[k * nk:(k + 1) * nk])
        outs.append(kern(emb_k, ids[k * nk:(k + 1) * nk], tab_i32,
                         gam, bet))
    return jnp.concatenate(outs).reshape(b, s, h)


# speedup vs baseline: 4.6114x; 4.6114x over previous
import functools

import jax
import jax.numpy as jnp
from jax import lax
from jax.experimental import pallas as pl
from jax.experimental.pallas import tpu as pltpu
from jax.experimental.pallas import tpu_sc as plsc

NC = 2
NS = 16
NW = NC * NS
L = 16
H = 768
HC = H // L
HW = H // 2
HC2 = H // (2 * L)
R = 32
NB = 2
EPS = 1e-12
MASK_HI = jnp.int32(-65536)


def _rsqrt(v):
    i = plsc.bitcast(v, jnp.int32)
    y = plsc.bitcast(jnp.int32(0x5F3759DF) - (i >> 1), jnp.float32)
    for _ in range(3):
        y = y * (1.5 - 0.5 * v * y * y)
    return y


def _make_kernel(n_rows):
    rows_per_w = n_rows // NW
    chunks = rows_per_w // R
    mesh = plsc.VectorSubcoreMesh(
        core_axis_name="c", subcore_axis_name="s",
        num_cores=NC, num_subcores=NS)

    @functools.partial(
        pl.kernel,
        out_type=jax.ShapeDtypeStruct((n_rows, H), jnp.float32),
        mesh=mesh,
        compiler_params=pltpu.CompilerParams(needs_layout_passes=False),
        scratch_types=[
            pltpu.VMEM((rows_per_w,), jnp.int32),
            pltpu.VMEM((NB, R, HW), jnp.int32),
            pltpu.VMEM((NB, R, H), jnp.float32),
            pltpu.VMEM((NB, R, H), jnp.float32),
            pltpu.VMEM((R * L,), jnp.float32),
            pltpu.VMEM((R * L,), jnp.float32),
            pltpu.SMEM((R,), jnp.float32),
            pltpu.SMEM((R,), jnp.float32),
            pltpu.VMEM((H,), jnp.float32),
            pltpu.VMEM((H,), jnp.float32),
            pltpu.SemaphoreType.DMA((NB,)),
            pltpu.SemaphoreType.DMA((NB,)),
            pltpu.SemaphoreType.DMA((NB,)),
            pltpu.SemaphoreType.DMA,
        ],
    )
    def kern(emb_hbm, ids_hbm, tab_hbm, gam_hbm, bet_hbm, out_hbm,
             ids_v, p_v, y_v, o_v, sp_v, sq_v, a_sm, d_sm, g_v, b_v,
             sem_g, sem_e, sem_o, sem_i):
        wid = lax.axis_index("s") * NC + lax.axis_index("c")
        wbase = wid * rows_per_w
        pltpu.sync_copy(gam_hbm, g_v)
        pltpu.sync_copy(bet_hbm, b_v)
        pltpu.async_copy(ids_hbm.at[pl.ds(wbase, rows_per_w)], ids_v,
                         sem_i).wait()

        def start_loads(c, nb):
            idx = ids_v.at[pl.ds(c * R, R)]
            pltpu.async_copy(tab_hbm.at[idx], p_v.at[nb], sem_g.at[nb])
            pltpu.async_copy(emb_hbm.at[pl.ds(wbase + c * R, R)],
                             y_v.at[nb], sem_e.at[nb])

        start_loads(0, 0)

        def chunk_body(c, _):
            nb = lax.rem(c, NB)

            idx = ids_v.at[pl.ds(c * R, R)]
            pltpu.make_async_copy(tab_hbm.at[idx], p_v.at[nb],
                                  sem_g.at[nb]).wait()
            pltpu.make_async_copy(emb_hbm.at[pl.ds(wbase + c * R, R)],
                                  y_v.at[nb], sem_e.at[nb]).wait()

            @pl.when(c + 1 < chunks)
            def _():
                start_loads(c + 1, lax.rem(c + 1, NB))

            @pl.when(c >= NB)
            def _():
                pltpu.make_async_copy(
                    o_v.at[nb],
                    out_hbm.at[pl.ds(wbase + (c - NB) * R, R)],
                    sem_o.at[nb]).wait()

            RI = 2
            def row_body(q, _):
                r0 = q * RI
                def h_body(m, carry):
                    out = []
                    for i in range(RI):
                        s, ss = carry[2 * i], carry[2 * i + 1]
                        pw = p_v[nb, r0 + i, pl.ds(m * L, L)]
                        lo = plsc.bitcast(pw << 16, jnp.float32)
                        hi = plsc.bitcast(pw & MASK_HI, jnp.float32)
                        sl0 = pl.ds(m * 2 * L, L)
                        sl1 = pl.ds(m * 2 * L + L, L)
                        x0 = y_v[nb, r0 + i, sl0] + lo
                        x1 = y_v[nb, r0 + i, sl1] + hi
                        o_v[nb, r0 + i, sl0] = x0
                        o_v[nb, r0 + i, sl1] = x1
                        out += [s + x0 + x1, ss + x0 * x0 + x1 * x1]
                    return tuple(out)
                z = jnp.zeros((L,), jnp.float32)
                acc = plsc.parallel_loop(
                    0, HC2, 1, unroll=4, carry=(z,) * (2 * RI))(h_body)
                for i in range(RI):
                    sp_v[pl.ds((r0 + i) * L, L)] = acc[2 * i]
                    sq_v[pl.ds((r0 + i) * L, L)] = acc[2 * i + 1]
                return 0
            lax.fori_loop(0, R // RI, row_body, 0)

            for k in range(R // L):
                rows16 = (lax.iota(jnp.int32, L) + k * L) * L
                s = jnp.zeros((L,), jnp.float32)
                ss = jnp.zeros((L,), jnp.float32)
                for j in range(L):
                    fidx = rows16 + j
                    s = s + plsc.load_gather(sp_v, [fidx])
                    ss = ss + plsc.load_gather(sq_v, [fidx])
                mean = s * (1.0 / H)
                var = ss * (1.0 / H) - mean * mean
                rstd = _rsqrt(var + EPS)
                nmr = -mean * rstd
                for j in range(L):
                    a_sm[k * L + j] = rstd[j]
                    d_sm[k * L + j] = nmr[j]

            def hb(h, _):
                sl = pl.ds(h * L, L)
                g = g_v[sl]
                b = b_v[sl]
                def rb(r):
                    x = o_v[nb, r, sl]
                    o_v[nb, r, sl] = (x * a_sm[r] + d_sm[r]) * g + b
                plsc.parallel_loop(0, R, 1, unroll=8)(rb)
                return 0
            lax.fori_loop(0, HC, hb, 0)

            pltpu.async_copy(o_v.at[nb],
                             out_hbm.at[pl.ds(wbase + c * R, R)],
                             sem_o.at[nb])
            return 0

        lax.fori_loop(0, chunks, chunk_body, 0)

        for j in range(NB):
            pltpu.make_async_copy(o_v.at[j], out_hbm.at[pl.ds(wbase, R)],
                                  sem_o.at[j]).wait()

    return kern


def kernel(inputs_embeds, position_ids, pos_table, ln_gamma, ln_beta):
    b, s, h = inputs_embeds.shape
    n = b * s
    emb = inputs_embeds.reshape(n, h)
    ids = position_ids.reshape(n).astype(jnp.int32)
    maxp = pos_table.shape[0]
    tab_u16 = lax.bitcast_convert_type(
        pos_table.astype(jnp.bfloat16), jnp.uint16)
    t3 = tab_u16.reshape(maxp, h // 32, 32).astype(jnp.int32)
    tab_i32 = (t3[:, :, :16] | (t3[:, :, 16:] << 16)).reshape(maxp, h // 2)
    out = _make_kernel(n)(emb, ids, tab_i32,
                          ln_gamma.astype(jnp.float32),
                          ln_beta.astype(jnp.float32))
    return out.reshape(b, s, h)


# speedup vs baseline: 5.2576x; 1.1401x over previous
import functools

import jax
import jax.numpy as jnp
from jax import lax
from jax.experimental import pallas as pl
from jax.experimental.pallas import tpu as pltpu
from jax.experimental.pallas import tpu_sc as plsc

NC = 2
NS = 16
NW = NC * NS
L = 16
H = 768
HC = H // L
HW = H // 2
HC2 = H // (2 * L)
R = 32
NB = 2
EPS = 1e-12
MASK_HI = jnp.int32(-65536)


def _rsqrt(v):
    i = plsc.bitcast(v, jnp.int32)
    y = plsc.bitcast(jnp.int32(0x5F3759DF) - (i >> 1), jnp.float32)
    for _ in range(3):
        y = y * (1.5 - 0.5 * v * y * y)
    return y


def _make_kernel(n_rows):
    rows_per_w = n_rows // NW
    chunks = rows_per_w // R
    mesh = plsc.VectorSubcoreMesh(
        core_axis_name="c", subcore_axis_name="s",
        num_cores=NC, num_subcores=NS)

    @functools.partial(
        pl.kernel,
        out_type=jax.ShapeDtypeStruct((n_rows, H), jnp.float32),
        mesh=mesh,
        compiler_params=pltpu.CompilerParams(needs_layout_passes=False),
        scratch_types=[
            pltpu.VMEM((rows_per_w,), jnp.int32),
            pltpu.VMEM((NB, R, HW), jnp.int32),
            pltpu.VMEM((NB, R, H), jnp.float32),
            pltpu.VMEM((NB, R, H), jnp.float32),
            pltpu.VMEM((R * L,), jnp.float32),
            pltpu.VMEM((R * L,), jnp.float32),
            pltpu.SMEM((R,), jnp.float32),
            pltpu.SMEM((R,), jnp.float32),
            pltpu.VMEM((H,), jnp.float32),
            pltpu.VMEM((H,), jnp.float32),
            pltpu.SemaphoreType.DMA((NB,)),
            pltpu.SemaphoreType.DMA((NB,)),
            pltpu.SemaphoreType.DMA((NB,)),
            pltpu.SemaphoreType.DMA,
        ],
    )
    def kern(emb_hbm, ids_hbm, tab_hbm, gam_hbm, bet_hbm, out_hbm,
             ids_v, p_v, y_v, o_v, sp_v, sq_v, a_sm, d_sm, g_v, b_v,
             sem_g, sem_e, sem_o, sem_i):
        wid = lax.axis_index("s") * NC + lax.axis_index("c")
        wbase = wid * rows_per_w
        pltpu.sync_copy(gam_hbm, g_v)
        pltpu.sync_copy(bet_hbm, b_v)
        pltpu.async_copy(ids_hbm.at[pl.ds(wbase, rows_per_w)], ids_v,
                         sem_i).wait()

        def start_loads(c, nb):
            idx = ids_v.at[pl.ds(c * R, R)]
            pltpu.async_copy(tab_hbm.at[idx], p_v.at[nb], sem_g.at[nb])
            pltpu.async_copy(emb_hbm.at[pl.ds(wbase + c * R, R)],
                             y_v.at[nb], sem_e.at[nb])

        start_loads(0, 0)

        def chunk_body(c, _):
            nb = lax.rem(c, NB)

            idx = ids_v.at[pl.ds(c * R, R)]
            pltpu.make_async_copy(tab_hbm.at[idx], p_v.at[nb],
                                  sem_g.at[nb]).wait()
            pltpu.make_async_copy(emb_hbm.at[pl.ds(wbase + c * R, R)],
                                  y_v.at[nb], sem_e.at[nb]).wait()

            @pl.when(c + 1 < chunks)
            def _():
                start_loads(c + 1, lax.rem(c + 1, NB))

            @pl.when(c >= NB)
            def _():
                pltpu.make_async_copy(
                    o_v.at[nb],
                    out_hbm.at[pl.ds(wbase + (c - NB) * R, R)],
                    sem_o.at[nb]).wait()

            RI = 2
            def row_body(q, _):
                r0 = q * RI
                def h_body(m, carry):
                    out = []
                    for i in range(RI):
                        s, ss = carry[2 * i], carry[2 * i + 1]
                        pw = p_v[nb, r0 + i, pl.ds(m * L, L)]
                        lo = plsc.bitcast(pw << 16, jnp.float32)
                        hi = plsc.bitcast(pw & MASK_HI, jnp.float32)
                        sl0 = pl.ds(m * L, L)
                        sl1 = pl.ds(HW + m * L, L)
                        x0 = y_v[nb, r0 + i, sl0] + lo
                        x1 = y_v[nb, r0 + i, sl1] + hi
                        o_v[nb, r0 + i, sl0] = x0
                        o_v[nb, r0 + i, sl1] = x1
                        out += [s + x0 + x1, ss + x0 * x0 + x1 * x1]
                    return tuple(out)
                z = jnp.zeros((L,), jnp.float32)
                acc = plsc.parallel_loop(
                    0, HC2, 1, unroll=4, carry=(z,) * (2 * RI))(h_body)
                for i in range(RI):
                    sp_v[pl.ds((r0 + i) * L, L)] = acc[2 * i]
                    sq_v[pl.ds((r0 + i) * L, L)] = acc[2 * i + 1]
                return 0
            lax.fori_loop(0, R // RI, row_body, 0)

            for k in range(R // L):
                rows16 = (lax.iota(jnp.int32, L) + k * L) * L
                s = jnp.zeros((L,), jnp.float32)
                ss = jnp.zeros((L,), jnp.float32)
                for j in range(L):
                    fidx = rows16 + j
                    s = s + plsc.load_gather(sp_v, [fidx])
                    ss = ss + plsc.load_gather(sq_v, [fidx])
                mean = s * (1.0 / H)
                var = ss * (1.0 / H) - mean * mean
                rstd = _rsqrt(var + EPS)
                nmr = -mean * rstd
                for j in range(L):
                    a_sm[k * L + j] = rstd[j]
                    d_sm[k * L + j] = nmr[j]

            def hb(h, _):
                sl = pl.ds(h * L, L)
                g = g_v[sl]
                b = b_v[sl]
                def rb(r):
                    x = o_v[nb, r, sl]
                    o_v[nb, r, sl] = (x * a_sm[r] + d_sm[r]) * g + b
                plsc.parallel_loop(0, R, 1, unroll=8)(rb)
                return 0
            lax.fori_loop(0, HC, hb, 0)

            pltpu.async_copy(o_v.at[nb],
                             out_hbm.at[pl.ds(wbase + c * R, R)],
                             sem_o.at[nb])
            return 0

        lax.fori_loop(0, chunks, chunk_body, 0)

        for j in range(NB):
            pltpu.make_async_copy(o_v.at[j], out_hbm.at[pl.ds(wbase, R)],
                                  sem_o.at[j]).wait()

    return kern


def kernel(inputs_embeds, position_ids, pos_table, ln_gamma, ln_beta):
    b, s, h = inputs_embeds.shape
    n = b * s
    emb = inputs_embeds.reshape(n, h)
    ids = position_ids.reshape(n).astype(jnp.int32)
    tab_u16 = lax.bitcast_convert_type(
        pos_table.astype(jnp.bfloat16), jnp.uint16)
    t32 = tab_u16.astype(jnp.int32)
    tab_i32 = t32[:, :h // 2] | (t32[:, h // 2:] << 16)
    out = _make_kernel(n)(emb, ids, tab_i32,
                          ln_gamma.astype(jnp.float32),
                          ln_beta.astype(jnp.float32))
    return out.reshape(b, s, h)
